# baseline (speedup 1.0000x reference)
import jax
import jax.numpy as jnp
from jax import lax
from jax.experimental import pallas as pl
from jax.experimental.pallas import tpu as pltpu
from jax.experimental.pallas import tpu_sc as plsc

_NC = 2
_NS = 16
_NW = _NC * _NS
_CHUNK = 80


def _mm_body(x_ref, w_ref, o_ref):
    o_ref[...] = jnp.dot(x_ref[...], w_ref[...],
                         preferred_element_type=jnp.float32)


def _matmul(x, w):
    return pl.pallas_call(
        _mm_body,
        out_shape=jax.ShapeDtypeStruct((x.shape[0], w.shape[1]), jnp.float32),
    )(x, w)


def _epi_body(p_ref, gb_ref, w_ref, lb_ref, o_ref):
    h = jnp.maximum(p_ref[0] + p_ref[1] + gb_ref[...], 0.0)
    o_ref[...] = lax.dot_general(
        h, w_ref[...], (((1,), (1,)), ((), ())),
        preferred_element_type=jnp.float32) + lb_ref[...]


def _epilogue(parts, gcn_bias, lin_weight, lin_bias):
    n, d = parts.shape[1], parts.shape[2]
    return pl.pallas_call(
        _epi_body,
        out_shape=jax.ShapeDtypeStruct((n, d), jnp.float32),
    )(parts, gcn_bias.reshape(1, d), lin_weight, lin_bias.reshape(1, d))


def _make_sc_scatter(n_pad, e, d):
    assert e % (_NW * _CHUNK) == 0 and n_pad % (_NS * 8) == 0
    assert _CHUNK % 8 == 0
    edges_per_w = e // _NW
    n_chunks = edges_per_w // _CHUNK
    assert n_chunks >= 8
    rows_per_tile = n_pad // _NS
    n_vreg = d // 16

    def body(sup, srcs, dsts, adjs, parts,
             acc, adj_slab, rows0, rows1, rows2, src0, src1, src2,
             dst0, dst1, dst2,
             gsem0, gsem1, gsem2, isem0, isem1, isem2,
             dsem0, dsem1, dsem2, ssem):
        c = lax.axis_index("c")
        s = lax.axis_index("s")
        w = c * _NS + s
        base_w = w * edges_per_w

        rows = (rows0, rows1, rows2)
        srcb = (src0, src1, src2)
        dstb = (dst0, dst1, dst2)
        gsems = (gsem0, gsem1, gsem2)
        isems = (isem0, isem1, isem2)
        dsems = (dsem0, dsem1, dsem2)

        def zrow_body(i, carry):
            for r in range(n_vreg):
                rows0[i, pl.ds(r * 16, 16)] = jnp.zeros((16,), jnp.float32)
            return carry

        lax.fori_loop(0, _CHUNK, zrow_body, 0)
        zbase = s * rows_per_tile
        n_zfull = rows_per_tile // _CHUNK
        zrem = rows_per_tile - n_zfull * _CHUNK

        def zissue(j, carry):
            pltpu.async_copy(
                rows0, acc.at[pl.ds(zbase + j * _CHUNK, _CHUNK)], gsem0)
            return carry

        lax.fori_loop(0, n_zfull, zissue, 0)
        if zrem:
            pltpu.async_copy(
                rows0.at[pl.ds(0, zrem)],
                acc.at[pl.ds(zbase + n_zfull * _CHUNK, zrem)], gsem1)

        def zdrain(j, carry):
            pltpu.make_async_copy(
                rows0, acc.at[pl.ds(zbase, _CHUNK)], gsem0).wait()
            return carry

        lax.fori_loop(0, n_zfull, zdrain, 0)
        if zrem:
            pltpu.make_async_copy(
                rows0.at[pl.ds(0, zrem)],
                acc.at[pl.ds(zbase, zrem)], gsem1).wait()

        pltpu.sync_copy(adjs.at[pl.ds(base_w, edges_per_w)],
                        adj_slab.at[pl.ds(0, edges_per_w)])
        pltpu.sync_copy(srcs.at[pl.ds(base_w, _CHUNK)], src0)
        pltpu.sync_copy(srcs.at[pl.ds(base_w + _CHUNK, _CHUNK)], src1)
        pltpu.async_copy(srcs.at[pl.ds(base_w + 2 * _CHUNK, _CHUNK)],
                         src2, isem2)
        pltpu.async_copy(dsts.at[pl.ds(base_w, _CHUNK)], dst0, dsem0)
        pltpu.async_copy(dsts.at[pl.ds(base_w + _CHUNK, _CHUNK)],
                         dst1, dsem1)
        plsc.subcore_barrier()

        def scale_rows(k, rv):
            @plsc.parallel_loop(0, _CHUNK, 8, unroll=10)
            def group_body(i):
                a_vec = adj_slab[pl.ds(k * _CHUNK + i, 16)]
                for t in range(8):
                    scale = lax.gather(
                        a_vec,
                        jnp.full((16, 1), t, dtype=jnp.int32),
                        lax.GatherDimensionNumbers(
                            offset_dims=(), collapsed_slice_dims=(0,),
                            start_index_map=(0,)),
                        (1,),
                        mode=lax.GatherScatterMode.PROMISE_IN_BOUNDS)
                    for r in range(n_vreg):
                        sl = pl.ds(r * 16, 16)
                        rv[i + t, sl] = rv[i + t, sl] * scale

        def issue_src(k, b):
            pltpu.async_copy(srcs.at[pl.ds(base_w + k * _CHUNK, _CHUNK)],
                             srcb[b], isems[b])

        def issue_dst(k, b):
            pltpu.async_copy(dsts.at[pl.ds(base_w + k * _CHUNK, _CHUNK)],
                             dstb[b], dsems[b])

        def wait_src(b):
            pltpu.make_async_copy(srcs.at[pl.ds(base_w, _CHUNK)],
                                  srcb[b], isems[b]).wait()

        def wait_dst(b):
            pltpu.make_async_copy(dsts.at[pl.ds(base_w, _CHUNK)],
                                  dstb[b], dsems[b]).wait()

        def issue_gather(b):
            pltpu.async_copy(sup.at[srcb[b]], rows[b], gsems[b])

        def wait_gather(b):
            pltpu.make_async_copy(sup.at[srcb[b]], rows[b], gsems[b]).wait()

        def wait_scatter(b):
            pltpu.make_async_copy(rows[b], acc.at[dstb[b]], ssem).wait()

        def step(j, b, first=False, pre_g=True, pre_s=True):
            if not first:
                wait_scatter((b + 2) % 3)
            if pre_g:
                b2 = (b + 2) % 3
                issue_dst(j + 2, b2)
                wait_src(b2)
                issue_gather(b2)
            wait_gather(b)
            if pre_s:
                issue_src(j + 3, b)
            scale_rows(j, rows[b])
            wait_dst(b)
            pltpu.async_copy(rows[b], acc.at[dstb[b]], ssem, add=True)

        issue_gather(0)
        issue_gather(1)
        step(0, 0, first=True)

        n_main = (n_chunks - 4) // 3 * 3

        @pl.loop(1, 1 + n_main, step=3)
        def chunk_loop(k):
            for off in range(3):
                j = k + off
                step(j, (1 + off) % 3)

        for j in range(1 + n_main, n_chunks):
            step(j, j % 3,
                 pre_g=(j + 2 < n_chunks), pre_s=(j + 3 < n_chunks))
        wait_scatter((n_chunks - 1) % 3)

        plsc.subcore_barrier()

        out_base = s * rows_per_tile
        n_pieces = rows_per_tile // 8

        def out_issue(j, carry):
            pltpu.async_copy(acc.at[pl.ds(out_base + j * 8, 8)],
                             parts.at[c, pl.ds(out_base + j * 8, 8)],
                             gsem0)
            return carry

        lax.fori_loop(0, n_pieces, out_issue, 0)

        def out_drain(j, carry):
            pltpu.make_async_copy(acc.at[pl.ds(out_base, 8)],
                                  parts.at[c, pl.ds(out_base, 8)],
                                  gsem0).wait()
            return carry

        lax.fori_loop(0, n_pieces, out_drain, 0)

    mesh = plsc.VectorSubcoreMesh(core_axis_name="c", subcore_axis_name="s",
                                  num_cores=_NC, num_subcores=_NS)
    return pl.kernel(
        body,
        out_type=jax.ShapeDtypeStruct((_NC, n_pad, d), jnp.float32),
        mesh=mesh,
        scratch_types=[
            pltpu.VMEM_SHARED((n_pad, d), jnp.float32),
            pltpu.VMEM((edges_per_w + 16,), jnp.float32),
            pltpu.VMEM((_CHUNK, d), jnp.float32),
            pltpu.VMEM((_CHUNK, d), jnp.float32),
            pltpu.VMEM((_CHUNK, d), jnp.float32),
            pltpu.VMEM((_CHUNK,), jnp.int32),
            pltpu.VMEM((_CHUNK,), jnp.int32),
            pltpu.VMEM((_CHUNK,), jnp.int32),
            pltpu.VMEM((_CHUNK,), jnp.int32),
            pltpu.VMEM((_CHUNK,), jnp.int32),
            pltpu.VMEM((_CHUNK,), jnp.int32),
            pltpu.SemaphoreType.DMA,
            pltpu.SemaphoreType.DMA,
            pltpu.SemaphoreType.DMA,
            pltpu.SemaphoreType.DMA,
            pltpu.SemaphoreType.DMA,
            pltpu.SemaphoreType.DMA,
            pltpu.SemaphoreType.DMA,
            pltpu.SemaphoreType.DMA,
            pltpu.SemaphoreType.DMA,
            pltpu.SemaphoreType.DMA,
        ],
    )


def kernel(input, edge_index, adj_values, gcn_weight, gcn_bias,
           lin_weight, lin_bias):
    x = input.astype(jnp.float32)
    n, d = x.shape
    e = adj_values.shape[0]
    dst = edge_index[0].astype(jnp.int32)
    src = edge_index[1].astype(jnp.int32)

    support = _matmul(x, gcn_weight)
    n_pad = -(-n // (_NS * 8)) * (_NS * 8)
    parts = _make_sc_scatter(n_pad, e, d)(support, src, dst,
                                          adj_values.astype(jnp.float32))
    return _epilogue(parts[:, :n, :], gcn_bias, lin_weight, lin_bias)


# speedup vs baseline: 1.0785x; 1.0785x over previous
import jax
import jax.numpy as jnp
from jax import lax
from jax.experimental import pallas as pl
from jax.experimental.pallas import tpu as pltpu
from jax.experimental.pallas import tpu_sc as plsc

_NC = 2
_NS = 16
_NW = _NC * _NS
_CHUNK = 80


def _mm_body(x_ref, w_ref, o_ref):
    o_ref[...] = jnp.dot(x_ref[...], w_ref[...],
                         preferred_element_type=jnp.float32)


def _matmul(x, w):
    return pl.pallas_call(
        _mm_body,
        out_shape=jax.ShapeDtypeStruct((x.shape[0], w.shape[1]), jnp.float32),
    )(x, w)


def _make_epi_body(n):
    def _epi_body(p_ref, gb_ref, w_ref, lb_ref, o_ref):
        h = jnp.maximum(p_ref[0, :n] + p_ref[1, :n] + gb_ref[...], 0.0)
        o_ref[...] = lax.dot_general(
            h, w_ref[...], (((1,), (1,)), ((), ())),
            preferred_element_type=jnp.float32) + lb_ref[...]
    return _epi_body


def _epilogue(parts, n, gcn_bias, lin_weight, lin_bias):
    d = parts.shape[2]
    return pl.pallas_call(
        _make_epi_body(n),
        out_shape=jax.ShapeDtypeStruct((n, d), jnp.float32),
    )(parts, gcn_bias.reshape(1, d), lin_weight, lin_bias.reshape(1, d))


def _make_sc_scatter(n_pad, e, d):
    assert e % (_NW * _CHUNK) == 0 and n_pad % (_NS * 8) == 0
    assert _CHUNK % 8 == 0
    edges_per_w = e // _NW
    n_chunks = edges_per_w // _CHUNK
    assert n_chunks >= 8
    rows_per_tile = n_pad // _NS
    n_vreg = d // 16

    def body(sup, srcs, dsts, adjs, parts,
             acc, adj_slab, rows0, rows1, rows2, src0, src1, src2,
             dst0, dst1, dst2,
             gsem0, gsem1, gsem2, isem0, isem1, isem2,
             dsem0, dsem1, dsem2, ssem):
        c = lax.axis_index("c")
        s = lax.axis_index("s")
        w = c * _NS + s
        base_w = w * edges_per_w

        rows = (rows0, rows1, rows2)
        srcb = (src0, src1, src2)
        dstb = (dst0, dst1, dst2)
        gsems = (gsem0, gsem1, gsem2)
        isems = (isem0, isem1, isem2)
        dsems = (dsem0, dsem1, dsem2)

        def zrow_body(i, carry):
            for r in range(n_vreg):
                rows0[i, pl.ds(r * 16, 16)] = jnp.zeros((16,), jnp.float32)
            return carry

        lax.fori_loop(0, _CHUNK, zrow_body, 0)
        zbase = s * rows_per_tile
        n_zfull = rows_per_tile // _CHUNK
        zrem = rows_per_tile - n_zfull * _CHUNK

        def zissue(j, carry):
            pltpu.async_copy(
                rows0, acc.at[pl.ds(zbase + j * _CHUNK, _CHUNK)], gsem0)
            return carry

        lax.fori_loop(0, n_zfull, zissue, 0)
        if zrem:
            pltpu.async_copy(
                rows0.at[pl.ds(0, zrem)],
                acc.at[pl.ds(zbase + n_zfull * _CHUNK, zrem)], gsem1)

        def zdrain(j, carry):
            pltpu.make_async_copy(
                rows0, acc.at[pl.ds(zbase, _CHUNK)], gsem0).wait()
            return carry

        lax.fori_loop(0, n_zfull, zdrain, 0)
        if zrem:
            pltpu.make_async_copy(
                rows0.at[pl.ds(0, zrem)],
                acc.at[pl.ds(zbase, zrem)], gsem1).wait()

        pltpu.sync_copy(adjs.at[pl.ds(base_w, edges_per_w)],
                        adj_slab.at[pl.ds(0, edges_per_w)])
        pltpu.sync_copy(srcs.at[pl.ds(base_w, _CHUNK)], src0)
        pltpu.sync_copy(srcs.at[pl.ds(base_w + _CHUNK, _CHUNK)], src1)
        pltpu.async_copy(srcs.at[pl.ds(base_w + 2 * _CHUNK, _CHUNK)],
                         src2, isem2)
        pltpu.async_copy(dsts.at[pl.ds(base_w, _CHUNK)], dst0, dsem0)
        pltpu.async_copy(dsts.at[pl.ds(base_w + _CHUNK, _CHUNK)],
                         dst1, dsem1)
        plsc.subcore_barrier()

        def scale_rows(k, rv):
            @plsc.parallel_loop(0, _CHUNK, 8, unroll=4)
            def group_body(i):
                a_vec = adj_slab[pl.ds(k * _CHUNK + i, 16)]
                for t in range(8):
                    scale = lax.gather(
                        a_vec,
                        jnp.full((16, 1), t, dtype=jnp.int32),
                        lax.GatherDimensionNumbers(
                            offset_dims=(), collapsed_slice_dims=(0,),
                            start_index_map=(0,)),
                        (1,),
                        mode=lax.GatherScatterMode.PROMISE_IN_BOUNDS)
                    for r in range(n_vreg):
                        sl = pl.ds(r * 16, 16)
                        rv[i + t, sl] = rv[i + t, sl] * scale

        def issue_src(k, b):
            pltpu.async_copy(srcs.at[pl.ds(base_w + k * _CHUNK, _CHUNK)],
                             srcb[b], isems[b])

        def issue_dst(k, b):
            pltpu.async_copy(dsts.at[pl.ds(base_w + k * _CHUNK, _CHUNK)],
                             dstb[b], dsems[b])

        def wait_src(b):
            pltpu.make_async_copy(srcs.at[pl.ds(base_w, _CHUNK)],
                                  srcb[b], isems[b]).wait()

        def wait_dst(b):
            pltpu.make_async_copy(dsts.at[pl.ds(base_w, _CHUNK)],
                                  dstb[b], dsems[b]).wait()

        def issue_gather(b):
            pltpu.async_copy(sup.at[srcb[b]], rows[b], gsems[b])

        def wait_gather(b):
            pltpu.make_async_copy(sup.at[srcb[b]], rows[b], gsems[b]).wait()

        def wait_scatter(b):
            pltpu.make_async_copy(rows[b], acc.at[dstb[b]], ssem).wait()

        def step(j, b, first=False, pre_g=True, pre_s=True):
            if not first:
                wait_scatter((b + 2) % 3)
            if pre_g:
                b2 = (b + 2) % 3
                issue_dst(j + 2, b2)
                wait_src(b2)
                issue_gather(b2)
            wait_gather(b)
            if pre_s:
                issue_src(j + 3, b)
            scale_rows(j, rows[b])
            wait_dst(b)
            pltpu.async_copy(rows[b], acc.at[dstb[b]], ssem, add=True)

        issue_gather(0)
        issue_gather(1)
        step(0, 0, first=True)

        n_main = (n_chunks - 4) // 3 * 3

        @pl.loop(1, 1 + n_main, step=3)
        def chunk_loop(k):
            for off in range(3):
                j = k + off
                step(j, (1 + off) % 3)

        for j in range(1 + n_main, n_chunks):
            step(j, j % 3,
                 pre_g=(j + 2 < n_chunks), pre_s=(j + 3 < n_chunks))
        wait_scatter((n_chunks - 1) % 3)

        plsc.subcore_barrier()

        out_base = s * rows_per_tile
        n_pieces = rows_per_tile // 8

        def out_issue(j, carry):
            pltpu.async_copy(acc.at[pl.ds(out_base + j * 8, 8)],
                             parts.at[c, pl.ds(out_base + j * 8, 8)],
                             gsem0)
            return carry

        lax.fori_loop(0, n_pieces, out_issue, 0)

        def out_drain(j, carry):
            pltpu.make_async_copy(acc.at[pl.ds(out_base, 8)],
                                  parts.at[c, pl.ds(out_base, 8)],
                                  gsem0).wait()
            return carry

        lax.fori_loop(0, n_pieces, out_drain, 0)

    mesh = plsc.VectorSubcoreMesh(core_axis_name="c", subcore_axis_name="s",
                                  num_cores=_NC, num_subcores=_NS)
    return pl.kernel(
        body,
        out_type=jax.ShapeDtypeStruct((_NC, n_pad, d), jnp.float32),
        mesh=mesh,
        scratch_types=[
            pltpu.VMEM_SHARED((n_pad, d), jnp.float32),
            pltpu.VMEM((edges_per_w + 16,), jnp.float32),
            pltpu.VMEM((_CHUNK, d), jnp.float32),
            pltpu.VMEM((_CHUNK, d), jnp.float32),
            pltpu.VMEM((_CHUNK, d), jnp.float32),
            pltpu.VMEM((_CHUNK,), jnp.int32),
            pltpu.VMEM((_CHUNK,), jnp.int32),
            pltpu.VMEM((_CHUNK,), jnp.int32),
            pltpu.VMEM((_CHUNK,), jnp.int32),
            pltpu.VMEM((_CHUNK,), jnp.int32),
            pltpu.VMEM((_CHUNK,), jnp.int32),
            pltpu.SemaphoreType.DMA,
            pltpu.SemaphoreType.DMA,
            pltpu.SemaphoreType.DMA,
            pltpu.SemaphoreType.DMA,
            pltpu.SemaphoreType.DMA,
            pltpu.SemaphoreType.DMA,
            pltpu.SemaphoreType.DMA,
            pltpu.SemaphoreType.DMA,
            pltpu.SemaphoreType.DMA,
            pltpu.SemaphoreType.DMA,
        ],
    )


def kernel(input, edge_index, adj_values, gcn_weight, gcn_bias,
           lin_weight, lin_bias):
    x = input.astype(jnp.float32)
    n, d = x.shape
    e = adj_values.shape[0]
    dst = edge_index[0].astype(jnp.int32)
    src = edge_index[1].astype(jnp.int32)

    support = _matmul(x, gcn_weight)
    n_pad = -(-n // (_NS * 8)) * (_NS * 8)
    parts = _make_sc_scatter(n_pad, e, d)(support, src, dst,
                                          adj_values.astype(jnp.float32))
    return _epilogue(parts, n, gcn_bias, lin_weight, lin_bias)
